# baseline (speedup 1.0000x reference)
import jax
import jax.numpy as jnp
from jax.experimental import pallas as pl
from jax.experimental.pallas import tpu as pltpu

EPS = 1e-09


def _stats_kernel(mask_ref, x_ref, tr_ref, trs_ref):
    a = x_ref[0, 0]
    m = mask_ref[0]
    md = m * m
    n = a.shape[0]
    ii = jax.lax.broadcasted_iota(jnp.int32, (n, n), 0)
    kk = jax.lax.broadcasted_iota(jnp.int32, (n, n), 1)
    diag = jnp.where(ii == kk, a, 0.0)
    tr_ref[0, 0] = jnp.sum(diag * md, keepdims=True)
    at = jnp.transpose(a)
    trs_ref[0, 0] = jnp.sum(a * at * md, keepdims=True)


def _norm_kernel(mask_ref, tr_ref, trs_ref, w_ref, wexp_ref, wbias_ref,
                 bias_ref, x_ref, o_ref):
    f = pl.program_id(1)
    nf = tr_ref.shape[2]
    a = x_ref[0, 0]
    n_sz = a.shape[0]

    m = mask_ref[0]
    md = m * m
    n = jnp.sum(md)
    n2 = jnp.maximum(n - 1.0, 1.0)
    trrow = tr_ref[0]
    trsrow = trs_ref[0]
    mean_b = jnp.sum(trrow) / (n * nf)
    var_b = (jnp.sum(trsrow) - jnp.sum(trrow * trrow) / n) / (n2 * nf)
    inv = jax.lax.rsqrt(var_b + EPS)

    fsel = jax.lax.broadcasted_iota(jnp.int32, (1, nf), 1) == f
    scale_vec = w_ref[...] * jnp.exp(wexp_ref[...]) + wbias_ref[...]
    scale = jnp.sum(jnp.where(fsel, scale_vec, 0.0))
    bias_f = jnp.sum(jnp.where(fsel, bias_ref[...], 0.0))

    ii = jax.lax.broadcasted_iota(jnp.int32, (n_sz, n_sz), 0)
    kk = jax.lax.broadcasted_iota(jnp.int32, (n_sz, n_sz), 1)
    eye = ii == kk
    m2 = jnp.transpose(m) * m
    centered = jnp.where(eye, a - mean_b, a) * m2
    o_ref[0, 0] = centered * (inv * scale) + jnp.where(eye, bias_f, 0.0)


def kernel(x, mask, weight, weight_exp, weight_bias, bias):
    b, f, n, _ = x.shape
    mask3 = mask.reshape(b, 1, n)
    tr4, trs4 = pl.pallas_call(
        _stats_kernel,
        grid=(b, f),
        in_specs=[
            pl.BlockSpec((1, 1, n), lambda i, j: (i, 0, 0)),
            pl.BlockSpec((1, 1, n, n), lambda i, j: (i, j, 0, 0)),
        ],
        out_specs=[
            pl.BlockSpec((1, 1, 1, 1), lambda i, j: (i, j, 0, 0)),
            pl.BlockSpec((1, 1, 1, 1), lambda i, j: (i, j, 0, 0)),
        ],
        out_shape=[
            jax.ShapeDtypeStruct((b, f, 1, 1), jnp.float32),
            jax.ShapeDtypeStruct((b, f, 1, 1), jnp.float32),
        ],
        compiler_params=pltpu.CompilerParams(
            dimension_semantics=("parallel", "arbitrary")),
    )(mask3, x)

    tr = tr4.reshape(b, 1, f)
    trs = trs4.reshape(b, 1, f)
    w2 = weight.reshape(1, f)
    wexp2 = weight_exp.reshape(1, f)
    wb2 = weight_bias.reshape(1, f)
    bias2 = bias.reshape(1, f)

    out = pl.pallas_call(
        _norm_kernel,
        grid=(b, f),
        in_specs=[
            pl.BlockSpec((1, 1, n), lambda i, j: (i, 0, 0)),
            pl.BlockSpec((1, 1, f), lambda i, j: (i, 0, 0)),
            pl.BlockSpec((1, 1, f), lambda i, j: (i, 0, 0)),
            pl.BlockSpec((1, f), lambda i, j: (0, 0)),
            pl.BlockSpec((1, f), lambda i, j: (0, 0)),
            pl.BlockSpec((1, f), lambda i, j: (0, 0)),
            pl.BlockSpec((1, f), lambda i, j: (0, 0)),
            pl.BlockSpec((1, 1, n, n), lambda i, j: (i, j, 0, 0)),
        ],
        out_specs=pl.BlockSpec((1, 1, n, n), lambda i, j: (i, j, 0, 0)),
        out_shape=jax.ShapeDtypeStruct((b, f, n, n), jnp.float32),
        compiler_params=pltpu.CompilerParams(
            dimension_semantics=("parallel", "arbitrary")),
    )(mask3, tr, trs, w2, wexp2, wb2, bias2, x)
    return out


# speedup vs baseline: 3.6481x; 3.6481x over previous
import jax
import jax.numpy as jnp
from jax.experimental import pallas as pl
from jax.experimental.pallas import tpu as pltpu

EPS = 1e-09
G = 8


def _stats_kernel(mask_ref, x_ref, tr_ref, trs_ref):
    n = x_ref.shape[2]
    m = mask_ref[0]
    md = m * m
    ii = jax.lax.broadcasted_iota(jnp.int32, (n, n), 0)
    kk = jax.lax.broadcasted_iota(jnp.int32, (n, n), 1)
    e2 = jnp.where(ii == kk, md, 0.0)
    for g in range(G):
        a = x_ref[0, g]
        at = jnp.transpose(a)
        tr_ref[0, g] = jnp.sum(a * e2, keepdims=True)
        trs_ref[0, g] = jnp.sum(a * at * md, keepdims=True)


def _norm_kernel(mask_ref, tr_ref, trs_ref, w_ref, wexp_ref, wbias_ref,
                 bias_ref, x_ref, o_ref):
    j = pl.program_id(0) % (tr_ref.shape[2] // G)
    nf = tr_ref.shape[2]
    n = x_ref.shape[2]

    m = mask_ref[0]
    md = m * m
    cnt = jnp.sum(md)
    cnt2 = jnp.maximum(cnt - 1.0, 1.0)
    trrow = tr_ref[0]
    trsrow = trs_ref[0]
    mean_b = jnp.sum(trrow) / (cnt * nf)
    var_b = (jnp.sum(trsrow) - jnp.sum(trrow * trrow) / cnt) / (cnt2 * nf)
    inv = jax.lax.rsqrt(var_b + EPS)

    scale_vec = w_ref[...] * jnp.exp(wexp_ref[...]) + wbias_ref[...]
    lane = jax.lax.broadcasted_iota(jnp.int32, (1, nf), 1)

    ii = jax.lax.broadcasted_iota(jnp.int32, (n, n), 0)
    kk = jax.lax.broadcasted_iota(jnp.int32, (n, n), 1)
    e1 = jnp.where(ii == kk, 1.0, 0.0)
    m2i = (jnp.transpose(m) * m) * inv
    me1 = mean_b * e1

    for g in range(G):
        fsel = lane == (j * G + g)
        s = jnp.sum(jnp.where(fsel, scale_vec, 0.0))
        bf = jnp.sum(jnp.where(fsel, bias_ref[...], 0.0))
        a = x_ref[0, g]
        o_ref[0, g] = ((a - me1) * m2i) * s + bf * e1


def kernel(x, mask, weight, weight_exp, weight_bias, bias):
    b, f, n, _ = x.shape
    fg = f // G
    mask3 = mask.reshape(b, 1, n)
    tr4, trs4 = pl.pallas_call(
        _stats_kernel,
        grid=(b * fg,),
        in_specs=[
            pl.BlockSpec((1, 1, n), lambda i: (i // fg, 0, 0)),
            pl.BlockSpec((1, G, n, n), lambda i: (i // fg, i % fg, 0, 0)),
        ],
        out_specs=[
            pl.BlockSpec((1, G, 1, 1), lambda i: (i // fg, i % fg, 0, 0)),
            pl.BlockSpec((1, G, 1, 1), lambda i: (i // fg, i % fg, 0, 0)),
        ],
        out_shape=[
            jax.ShapeDtypeStruct((b, f, 1, 1), jnp.float32),
            jax.ShapeDtypeStruct((b, f, 1, 1), jnp.float32),
        ],
        compiler_params=pltpu.CompilerParams(
            dimension_semantics=("parallel",)),
    )(mask3, x)

    tr = tr4.reshape(b, 1, f)
    trs = trs4.reshape(b, 1, f)
    w2 = weight.reshape(1, f)
    wexp2 = weight_exp.reshape(1, f)
    wb2 = weight_bias.reshape(1, f)
    bias2 = bias.reshape(1, f)

    out = pl.pallas_call(
        _norm_kernel,
        grid=(b * fg,),
        in_specs=[
            pl.BlockSpec((1, 1, n), lambda i: (i // fg, 0, 0)),
            pl.BlockSpec((1, 1, f), lambda i: (i // fg, 0, 0)),
            pl.BlockSpec((1, 1, f), lambda i: (i // fg, 0, 0)),
            pl.BlockSpec((1, f), lambda i: (0, 0)),
            pl.BlockSpec((1, f), lambda i: (0, 0)),
            pl.BlockSpec((1, f), lambda i: (0, 0)),
            pl.BlockSpec((1, f), lambda i: (0, 0)),
            pl.BlockSpec((1, G, n, n), lambda i: (i // fg, i % fg, 0, 0)),
        ],
        out_specs=pl.BlockSpec((1, G, n, n), lambda i: (i // fg, i % fg, 0, 0)),
        out_shape=jax.ShapeDtypeStruct((b, f, n, n), jnp.float32),
        compiler_params=pltpu.CompilerParams(
            dimension_semantics=("parallel",)),
    )(mask3, tr, trs, w2, wexp2, wb2, bias2, x)
    return out


# speedup vs baseline: 4.4291x; 1.2141x over previous
import jax
import jax.numpy as jnp
from jax.experimental import pallas as pl
from jax.experimental.pallas import tpu as pltpu

EPS = 1e-09
G = 16


def _stats_kernel(mask_ref, x_ref, tr_ref, trs_ref):
    n = x_ref.shape[2]
    m = mask_ref[0]
    md = m * m
    ii = jax.lax.broadcasted_iota(jnp.int32, (n, n), 0)
    kk = jax.lax.broadcasted_iota(jnp.int32, (n, n), 1)
    e2 = jnp.where(ii == kk, md, 0.0)
    for g in range(G):
        a = x_ref[0, g]
        at = jnp.transpose(a)
        tr_ref[0, g] = jnp.sum(a * e2, keepdims=True)
        trs_ref[0, g] = jnp.sum(a * at * md, keepdims=True)


def _norm_kernel(mask_ref, tr_ref, trs_ref, w_ref, wexp_ref, wbias_ref,
                 bias_ref, x_ref, o_ref):
    j = pl.program_id(0) % (tr_ref.shape[2] // G)
    nf = tr_ref.shape[2]
    n = x_ref.shape[2]

    m = mask_ref[0]
    md = m * m
    cnt = jnp.sum(md)
    cnt2 = jnp.maximum(cnt - 1.0, 1.0)
    trrow = tr_ref[0]
    trsrow = trs_ref[0]
    mean_b = jnp.sum(trrow) / (cnt * nf)
    var_b = (jnp.sum(trsrow) - jnp.sum(trrow * trrow) / cnt) / (cnt2 * nf)
    inv = jax.lax.rsqrt(var_b + EPS)

    scale_vec = w_ref[...] * jnp.exp(wexp_ref[...]) + wbias_ref[...]
    lane = jax.lax.broadcasted_iota(jnp.int32, (1, nf), 1)

    ii = jax.lax.broadcasted_iota(jnp.int32, (n, n), 0)
    kk = jax.lax.broadcasted_iota(jnp.int32, (n, n), 1)
    e1 = jnp.where(ii == kk, 1.0, 0.0)
    m2i = (jnp.transpose(m) * m) * inv
    me1 = mean_b * e1

    for g in range(G):
        fsel = lane == (j * G + g)
        s = jnp.sum(jnp.where(fsel, scale_vec, 0.0))
        bf = jnp.sum(jnp.where(fsel, bias_ref[...], 0.0))
        a = x_ref[0, g]
        o_ref[0, g] = ((a - me1) * m2i) * s + bf * e1


def kernel(x, mask, weight, weight_exp, weight_bias, bias):
    b, f, n, _ = x.shape
    fg = f // G
    mask3 = mask.reshape(b, 1, n)
    tr4, trs4 = pl.pallas_call(
        _stats_kernel,
        grid=(b * fg,),
        in_specs=[
            pl.BlockSpec((1, 1, n), lambda i: (i // fg, 0, 0)),
            pl.BlockSpec((1, G, n, n), lambda i: (i // fg, i % fg, 0, 0)),
        ],
        out_specs=[
            pl.BlockSpec((1, G, 1, 1), lambda i: (i // fg, i % fg, 0, 0)),
            pl.BlockSpec((1, G, 1, 1), lambda i: (i // fg, i % fg, 0, 0)),
        ],
        out_shape=[
            jax.ShapeDtypeStruct((b, f, 1, 1), jnp.float32),
            jax.ShapeDtypeStruct((b, f, 1, 1), jnp.float32),
        ],
        compiler_params=pltpu.CompilerParams(
            dimension_semantics=("parallel",)),
    )(mask3, x)

    tr = tr4.reshape(b, 1, f)
    trs = trs4.reshape(b, 1, f)
    w2 = weight.reshape(1, f)
    wexp2 = weight_exp.reshape(1, f)
    wb2 = weight_bias.reshape(1, f)
    bias2 = bias.reshape(1, f)

    out = pl.pallas_call(
        _norm_kernel,
        grid=(b * fg,),
        in_specs=[
            pl.BlockSpec((1, 1, n), lambda i: (i // fg, 0, 0)),
            pl.BlockSpec((1, 1, f), lambda i: (i // fg, 0, 0)),
            pl.BlockSpec((1, 1, f), lambda i: (i // fg, 0, 0)),
            pl.BlockSpec((1, f), lambda i: (0, 0)),
            pl.BlockSpec((1, f), lambda i: (0, 0)),
            pl.BlockSpec((1, f), lambda i: (0, 0)),
            pl.BlockSpec((1, f), lambda i: (0, 0)),
            pl.BlockSpec((1, G, n, n), lambda i: (i // fg, i % fg, 0, 0)),
        ],
        out_specs=pl.BlockSpec((1, G, n, n), lambda i: (i // fg, i % fg, 0, 0)),
        out_shape=jax.ShapeDtypeStruct((b, f, n, n), jnp.float32),
        compiler_params=pltpu.CompilerParams(
            dimension_semantics=("parallel",)),
    )(mask3, tr, trs, w2, wexp2, wb2, bias2, x)
    return out


# speedup vs baseline: 4.8096x; 1.0859x over previous
import jax
import jax.numpy as jnp
from jax.experimental import pallas as pl
from jax.experimental.pallas import tpu as pltpu

EPS = 1e-09
G = 32


def _stats_kernel(mask_ref, x_ref, tr_ref, trs_ref):
    n = x_ref.shape[2]
    m = mask_ref[0]
    md = m * m
    ii = jax.lax.broadcasted_iota(jnp.int32, (n, n), 0)
    kk = jax.lax.broadcasted_iota(jnp.int32, (n, n), 1)
    e2 = jnp.where(ii == kk, md, 0.0)
    for g in range(G):
        a = x_ref[0, g]
        at = jnp.transpose(a)
        tr_ref[0, g] = jnp.sum(a * e2, keepdims=True)
        trs_ref[0, g] = jnp.sum(a * at * md, keepdims=True)


def _norm_kernel(mask_ref, tr_ref, trs_ref, w_ref, wexp_ref, wbias_ref,
                 bias_ref, x_ref, o_ref):
    j = pl.program_id(0) % (tr_ref.shape[2] // G)
    nf = tr_ref.shape[2]
    n = x_ref.shape[2]

    m = mask_ref[0]
    md = m * m
    cnt = jnp.sum(md)
    cnt2 = jnp.maximum(cnt - 1.0, 1.0)
    trrow = tr_ref[0]
    trsrow = trs_ref[0]
    mean_b = jnp.sum(trrow) / (cnt * nf)
    var_b = (jnp.sum(trsrow) - jnp.sum(trrow * trrow) / cnt) / (cnt2 * nf)
    inv = jax.lax.rsqrt(var_b + EPS)

    scale_vec = w_ref[...] * jnp.exp(wexp_ref[...]) + wbias_ref[...]
    lane = jax.lax.broadcasted_iota(jnp.int32, (1, nf), 1)

    ii = jax.lax.broadcasted_iota(jnp.int32, (n, n), 0)
    kk = jax.lax.broadcasted_iota(jnp.int32, (n, n), 1)
    e1 = jnp.where(ii == kk, 1.0, 0.0)
    m2i = (jnp.transpose(m) * m) * inv
    me1 = mean_b * e1

    for g in range(G):
        fsel = lane == (j * G + g)
        s = jnp.sum(jnp.where(fsel, scale_vec, 0.0))
        bf = jnp.sum(jnp.where(fsel, bias_ref[...], 0.0))
        a = x_ref[0, g]
        o_ref[0, g] = ((a - me1) * m2i) * s + bf * e1


def kernel(x, mask, weight, weight_exp, weight_bias, bias):
    b, f, n, _ = x.shape
    fg = f // G
    mask3 = mask.reshape(b, 1, n)
    tr4, trs4 = pl.pallas_call(
        _stats_kernel,
        grid=(b * fg,),
        in_specs=[
            pl.BlockSpec((1, 1, n), lambda i: (i // fg, 0, 0)),
            pl.BlockSpec((1, G, n, n), lambda i: (i // fg, i % fg, 0, 0)),
        ],
        out_specs=[
            pl.BlockSpec((1, G, 1, 1), lambda i: (i // fg, i % fg, 0, 0)),
            pl.BlockSpec((1, G, 1, 1), lambda i: (i // fg, i % fg, 0, 0)),
        ],
        out_shape=[
            jax.ShapeDtypeStruct((b, f, 1, 1), jnp.float32),
            jax.ShapeDtypeStruct((b, f, 1, 1), jnp.float32),
        ],
        compiler_params=pltpu.CompilerParams(
            dimension_semantics=("parallel",)),
    )(mask3, x)

    tr = tr4.reshape(b, 1, f)
    trs = trs4.reshape(b, 1, f)
    w2 = weight.reshape(1, f)
    wexp2 = weight_exp.reshape(1, f)
    wb2 = weight_bias.reshape(1, f)
    bias2 = bias.reshape(1, f)

    out = pl.pallas_call(
        _norm_kernel,
        grid=(b * fg,),
        in_specs=[
            pl.BlockSpec((1, 1, n), lambda i: (i // fg, 0, 0)),
            pl.BlockSpec((1, 1, f), lambda i: (i // fg, 0, 0)),
            pl.BlockSpec((1, 1, f), lambda i: (i // fg, 0, 0)),
            pl.BlockSpec((1, f), lambda i: (0, 0)),
            pl.BlockSpec((1, f), lambda i: (0, 0)),
            pl.BlockSpec((1, f), lambda i: (0, 0)),
            pl.BlockSpec((1, f), lambda i: (0, 0)),
            pl.BlockSpec((1, G, n, n), lambda i: (i // fg, i % fg, 0, 0)),
        ],
        out_specs=pl.BlockSpec((1, G, n, n), lambda i: (i // fg, i % fg, 0, 0)),
        out_shape=jax.ShapeDtypeStruct((b, f, n, n), jnp.float32),
        compiler_params=pltpu.CompilerParams(
            dimension_semantics=("parallel",)),
    )(mask3, tr, trs, w2, wexp2, wb2, bias2, x)
    return out
